# baseline (speedup 1.0000x reference)
import jax
import jax.numpy as jnp
from jax.experimental import pallas as pl
from jax.experimental.pallas import tpu as pltpu

_MAX_DELAY = 8
_TILE = 2048
_NBUF = 2


def _body(delay_ref, x_ref, pe_ref, off_ref, out_ref, scratch, sems):
    del delay_ref
    t = pl.program_id(0)
    b = pl.program_id(1)
    n_b = pl.num_programs(1)
    step = t * n_b + b
    buf = jax.lax.rem(step, _NBUF)

    @pl.when(step >= _NBUF)
    def _wait_prev():
        pltpu.make_async_copy(
            scratch.at[buf], out_ref.at[b, pl.ds(t * _TILE, _TILE), :], sems.at[buf]
        ).wait()

    scratch[buf] = x_ref[0] + pe_ref[...] + off_ref[0]
    pltpu.make_async_copy(
        scratch.at[buf], out_ref.at[b, pl.ds(t * _TILE, _TILE), :], sems.at[buf]
    ).start()

    @pl.when(step == pl.num_programs(0) * n_b - 1)
    def _drain():
        for k in range(_NBUF):
            pltpu.make_async_copy(
                scratch.at[k], out_ref.at[b, pl.ds(t * _TILE, _TILE), :], sems.at[k]
            ).wait()


def kernel(x, delay, offset_embed, sin_pe):
    B, T, D = x.shape
    pe = sin_pe[:T]
    off3 = offset_embed.reshape(offset_embed.shape[0], 1, D)
    n_t = T // _TILE

    grid_spec = pltpu.PrefetchScalarGridSpec(
        num_scalar_prefetch=1,
        grid=(n_t, B),
        in_specs=[
            pl.BlockSpec((1, _TILE, D), lambda t, b, d: (b, t, 0)),
            pl.BlockSpec((_TILE, D), lambda t, b, d: (t, 0)),
            pl.BlockSpec((1, 1, D), lambda t, b, d: (jnp.clip(d[b], 0, _MAX_DELAY), 0, 0)),
        ],
        out_specs=pl.BlockSpec(memory_space=pltpu.MemorySpace.HBM),
        scratch_shapes=[
            pltpu.VMEM((_NBUF, _TILE, D), jnp.float32),
            pltpu.SemaphoreType.DMA((_NBUF,)),
        ],
    )
    return pl.pallas_call(
        _body,
        grid_spec=grid_spec,
        out_shape=jax.ShapeDtypeStruct((B, T, D), x.dtype),
    )(delay, x, pe, off3)


# speedup vs baseline: 1.0038x; 1.0038x over previous
import jax
import jax.numpy as jnp
from jax.experimental import pallas as pl
from jax.experimental.pallas import tpu as pltpu

_MAX_DELAY = 8
_TILE = 2048
_HALF = _TILE // 2
_NSEM = 4


def _body(delay_ref, x_ref, pe_ref, off_ref, out_ref, scratch, sems):
    del delay_ref
    t = pl.program_id(0)
    b = pl.program_id(1)
    n_b = pl.num_programs(1)
    step = t * n_b + b
    for h in range(2):
        chunk = 2 * step + h
        slot = jax.lax.rem(chunk, _NSEM)
        row0 = t * _TILE + h * _HALF

        @pl.when(chunk >= _NSEM)
        def _wait_prev():
            pltpu.make_async_copy(
                scratch.at[slot], out_ref.at[b, pl.ds(row0, _HALF), :], sems.at[slot]
            ).wait()

        scratch[slot] = (
            x_ref[0, h * _HALF : (h + 1) * _HALF]
            + pe_ref[h * _HALF : (h + 1) * _HALF]
            + off_ref[0]
        )
        pltpu.make_async_copy(
            scratch.at[slot], out_ref.at[b, pl.ds(row0, _HALF), :], sems.at[slot]
        ).start()

    @pl.when(step == pl.num_programs(0) * n_b - 1)
    def _drain():
        for k in range(_NSEM):
            pltpu.make_async_copy(
                scratch.at[k], out_ref.at[b, pl.ds(t * _TILE, _HALF), :], sems.at[k]
            ).wait()


def kernel(x, delay, offset_embed, sin_pe):
    B, T, D = x.shape
    pe = sin_pe[:T]
    off3 = offset_embed.reshape(offset_embed.shape[0], 1, D)
    n_t = T // _TILE

    grid_spec = pltpu.PrefetchScalarGridSpec(
        num_scalar_prefetch=1,
        grid=(n_t, B),
        in_specs=[
            pl.BlockSpec((1, _TILE, D), lambda t, b, d: (b, t, 0)),
            pl.BlockSpec((_TILE, D), lambda t, b, d: (t, 0)),
            pl.BlockSpec((1, 1, D), lambda t, b, d: (jnp.clip(d[b], 0, _MAX_DELAY), 0, 0)),
        ],
        out_specs=pl.BlockSpec(memory_space=pltpu.MemorySpace.HBM),
        scratch_shapes=[
            pltpu.VMEM((_NSEM, _HALF, D), jnp.float32),
            pltpu.SemaphoreType.DMA((_NSEM,)),
        ],
    )
    return pl.pallas_call(
        _body,
        grid_spec=grid_spec,
        out_shape=jax.ShapeDtypeStruct((B, T, D), x.dtype),
    )(delay, x, pe, off3)


# speedup vs baseline: 1.0085x; 1.0048x over previous
import jax
import jax.numpy as jnp
from jax.experimental import pallas as pl
from jax.experimental.pallas import tpu as pltpu

_MAX_DELAY = 8
_TILE = 2048
_CHUNK = _TILE // 4
_NSEM = 8


def _body(delay_ref, x_ref, pe_ref, off_ref, out_ref, scratch, sems):
    del delay_ref
    t = pl.program_id(0)
    b = pl.program_id(1)
    n_b = pl.num_programs(1)
    step = t * n_b + b
    for h in range(4):
        chunk = 4 * step + h
        slot = jax.lax.rem(chunk, _NSEM)
        row0 = t * _TILE + h * _CHUNK

        @pl.when(chunk >= _NSEM)
        def _wait_prev():
            pltpu.make_async_copy(
                scratch.at[slot], out_ref.at[b, pl.ds(row0, _CHUNK), :], sems.at[slot]
            ).wait()

        scratch[slot] = (
            x_ref[0, h * _CHUNK : (h + 1) * _CHUNK]
            + pe_ref[h * _CHUNK : (h + 1) * _CHUNK]
            + off_ref[0]
        )
        pltpu.make_async_copy(
            scratch.at[slot], out_ref.at[b, pl.ds(row0, _CHUNK), :], sems.at[slot]
        ).start()

    @pl.when(step == pl.num_programs(0) * n_b - 1)
    def _drain():
        for k in range(_NSEM):
            pltpu.make_async_copy(
                scratch.at[k], out_ref.at[b, pl.ds(t * _TILE, _CHUNK), :], sems.at[k]
            ).wait()


def kernel(x, delay, offset_embed, sin_pe):
    B, T, D = x.shape
    pe = sin_pe[:T]
    off3 = offset_embed.reshape(offset_embed.shape[0], 1, D)
    n_t = T // _TILE

    grid_spec = pltpu.PrefetchScalarGridSpec(
        num_scalar_prefetch=1,
        grid=(n_t, B),
        in_specs=[
            pl.BlockSpec((1, _TILE, D), lambda t, b, d: (b, t, 0)),
            pl.BlockSpec((_TILE, D), lambda t, b, d: (t, 0)),
            pl.BlockSpec((1, 1, D), lambda t, b, d: (jnp.clip(d[b], 0, _MAX_DELAY), 0, 0)),
        ],
        out_specs=pl.BlockSpec(memory_space=pltpu.MemorySpace.HBM),
        scratch_shapes=[
            pltpu.VMEM((_NSEM, _CHUNK, D), jnp.float32),
            pltpu.SemaphoreType.DMA((_NSEM,)),
        ],
    )
    return pl.pallas_call(
        _body,
        grid_spec=grid_spec,
        out_shape=jax.ShapeDtypeStruct((B, T, D), x.dtype),
    )(delay, x, pe, off3)
